# baseline (device time: 114142 ns/iter reference)
import jax
import jax.numpy as jnp
from jax import lax
from jax.experimental import pallas as pl
from jax.experimental.pallas import tpu as pltpu

N_DEV = 16
SQ = 1024
D = 1024
HQ = 8
DH = 128
SCALE = 0.08838834764831843

RS_BITS = (0, 2, 1, 3)
AG_BITS = (3, 1, 2, 0)


def _bit(my, bi):
    return lax.bitwise_and(lax.shift_right_logical(my, bi), 1)


def _body(x_ref, wq_ref, wk_ref, wv_ref, wo_ref, cos_ref, sin_ref,
          out_ref, send_buf,
          rs_b0, rs_b1, rs_b2, rs_b3,
          rs_send, rs_recv, ag_send, ag_recv):
    my = lax.axis_index("i")

    f32 = jnp.float32
    bf16 = jnp.bfloat16
    mm = lambda a, b: lax.dot_general(
        a, b, (((1,), (0,)), ((), ())), preferred_element_type=f32)

    b = [_bit(my, RS_BITS[r]) for r in range(4)]
    partners = [(lax.bitwise_xor(my, 1 << RS_BITS[r]),) for r in range(4)]
    send0 = (1 - b[0]) * 512
    base0 = b[0] * 512
    s1 = base0 + (1 - b[1]) * 256
    base1 = base0 + b[1] * 256
    s2 = base1 + (1 - b[2]) * 128
    base2 = base1 + b[2] * 128
    s3 = base2 + (1 - b[3]) * 64
    base3 = base2 + b[3] * 64

    xv = x_ref[:, :]
    cos = cos_ref[:, :]
    sin = sin_ref[:, :]

    def rope(t, c, s):
        parts = []
        for h in range(HQ):
            lo = t[:, h * DH: h * DH + DH // 2]
            hi = t[:, h * DH + DH // 2: (h + 1) * DH]
            parts.append(-hi)
            parts.append(lo)
        return t * c + jnp.concatenate(parts, axis=1) * s

    kr = rope(mm(xv, wk_ref[:, :]).astype(bf16), cos, sin)
    v = mm(xv, wv_ref[:, :]).astype(bf16)

    def partial_rows(row_start, n):
        xr = x_ref[pl.ds(row_start, n), :]
        qb = mm(xr, wq_ref[:, :]).astype(bf16)
        cb = cos_ref[pl.ds(row_start, n), :]
        sb = sin_ref[pl.ds(row_start, n), :]
        qrb = rope(qb, cb, sb)
        acch = jnp.zeros((n, D), f32)
        for h in range(HQ):
            hs = slice(h * DH, (h + 1) * DH)
            s = lax.dot_general(
                qrb[:, hs], kr[:, hs], (((1,), (1,)), ((), ())),
                preferred_element_type=f32)
            w = jnp.exp(s * SCALE)
            w = w / jnp.sum(w, axis=-1, keepdims=True)
            ctx = mm(w.astype(bf16), v[:, hs]).astype(bf16)
            acch = acch + mm(ctx, wo_ref[hs, :])
        return acch

    def rs_rdma(r, half, dst):
        return pltpu.make_async_remote_copy(
            src_ref=send_buf.at[pl.ds(0, half), :],
            dst_ref=dst.at[:, :],
            send_sem=rs_send.at[r],
            recv_sem=rs_recv.at[r],
            device_id=partners[r],
            device_id_type=pl.DeviceIdType.MESH,
        )

    blk = partial_rows(send0, 512)
    send_buf[pl.ds(0, 512), :] = blk.astype(bf16)
    r0 = rs_rdma(0, 512, rs_b0)
    r0.start()

    blk1 = partial_rows(s1, 256)
    r0.wait()
    stage = blk1 + rs_b0[pl.ds(s1 - base0, 256), :].astype(f32)
    send_buf[pl.ds(0, 256), :] = stage.astype(bf16)
    r1 = rs_rdma(1, 256, rs_b1)
    r1.start()

    blk2 = partial_rows(s2, 128)
    r1.wait()
    stage = (blk2
             + rs_b0[pl.ds(s2 - base0, 128), :].astype(f32)
             + rs_b1[pl.ds(s2 - base1, 128), :].astype(f32))
    send_buf[pl.ds(0, 128), :] = stage.astype(bf16)
    r2 = rs_rdma(2, 128, rs_b2)
    r2.start()

    blk3 = partial_rows(s3, 64)
    r2.wait()
    stage = (blk3
             + rs_b0[pl.ds(s3 - base0, 64), :].astype(f32)
             + rs_b1[pl.ds(s3 - base1, 64), :].astype(f32)
             + rs_b2[pl.ds(s3 - base2, 64), :].astype(f32))
    send_buf[pl.ds(0, 64), :] = stage.astype(bf16)
    r3 = rs_rdma(3, 64, rs_b3)
    r3.start()

    own_blk = partial_rows(base3, 64)
    r3.wait()
    own = (own_blk
           + rs_b0[pl.ds(base3 - base0, 64), :].astype(f32)
           + rs_b1[pl.ds(base3 - base1, 64), :].astype(f32)
           + rs_b2[pl.ds(base3 - base2, 64), :].astype(f32)
           + rs_b3[:, :].astype(f32))
    out_ref[pl.ds(base3, 64), :] = own.astype(bf16)

    base = base3
    for r, bi in enumerate(AG_BITS):
        size = 64 << r
        bit = _bit(my, bi)
        rdma = pltpu.make_async_remote_copy(
            src_ref=out_ref.at[pl.ds(base, size), :],
            dst_ref=out_ref.at[pl.ds(base, size), :],
            send_sem=ag_send.at[r],
            recv_sem=ag_recv.at[r],
            device_id=(lax.bitwise_xor(my, 1 << bi),),
            device_id_type=pl.DeviceIdType.MESH,
        )
        rdma.start()
        rdma.wait()
        base = base - bit * size


def kernel(x, Wq, Wk, Wv, Wo):
    bf16 = jnp.bfloat16
    x2 = x.reshape(SQ, D).astype(bf16)

    def perm(w):
        return w.reshape(D, HQ, DH // 2, 2).transpose(0, 1, 3, 2).reshape(D, HQ * DH)

    wq = perm(Wq).astype(bf16)
    wk = perm(Wk).astype(bf16)
    wv = Wv.astype(bf16)
    wo = Wo.astype(bf16)

    inv = 1.0 / (10000.0 ** (jnp.arange(0, DH, 2, dtype=jnp.float32) / DH))
    pos = jnp.arange(SQ, dtype=jnp.float32)[:, None] * inv[None, :]
    cos_h = jnp.concatenate([jnp.cos(pos), jnp.cos(pos)], axis=-1)
    sin_h = jnp.concatenate([jnp.sin(pos), jnp.sin(pos)], axis=-1)
    cos_t = jnp.tile(cos_h, (1, HQ)).astype(bf16)
    sin_t = jnp.tile(sin_h, (1, HQ)).astype(bf16)

    out = pl.pallas_call(
        _body,
        out_shape=jax.ShapeDtypeStruct((SQ, D), bf16),
        in_specs=[pl.BlockSpec(memory_space=pltpu.VMEM)] * 7,
        out_specs=pl.BlockSpec(memory_space=pltpu.VMEM),
        scratch_shapes=[
            pltpu.VMEM((512, D), bf16),
            pltpu.VMEM((512, D), bf16),
            pltpu.VMEM((256, D), bf16),
            pltpu.VMEM((128, D), bf16),
            pltpu.VMEM((64, D), bf16),
            pltpu.SemaphoreType.DMA((4,)),
            pltpu.SemaphoreType.DMA((4,)),
            pltpu.SemaphoreType.DMA((4,)),
            pltpu.SemaphoreType.DMA((4,)),
        ],
    )(x2, wq, wk, wv, wo, cos_t, sin_t)
    return out.reshape(1, SQ, D).astype(jnp.float32)


# device time: 103806 ns/iter; 1.0996x vs baseline; 1.0996x over previous
import jax
import jax.numpy as jnp
from jax import lax
from jax.experimental import pallas as pl
from jax.experimental.pallas import tpu as pltpu

N_DEV = 16
SQ = 1024
D = 1024
HQ = 8
DH = 128
SCALE = 0.08838834764831843

RS_BITS = (0, 2, 1, 3)
AG_BITS = (3, 1, 2, 0)


def _bit(my, bi):
    return lax.bitwise_and(lax.shift_right_logical(my, bi), 1)


def _body(x_ref, wq_ref, wk_ref, wv_ref, wo_ref, cos_ref, sin_ref,
          out_ref, send_buf, dm_buf,
          rs_b0, rs_b1, rs_b2, rs_b3,
          rs_send, rs_recv, ag_send, ag_recv):
    my = lax.axis_index("i")

    f32 = jnp.float32
    bf16 = jnp.bfloat16
    mm = lambda a, b: lax.dot_general(
        a, b, (((1,), (0,)), ((), ())), preferred_element_type=f32)

    b = [_bit(my, bi) for bi in RS_BITS]
    rs_partners = [(lax.bitwise_xor(my, 1 << bi),) for bi in RS_BITS]
    send0 = (1 - b[0]) * 512
    base0 = b[0] * 512
    s1 = base0 + (1 - b[1]) * 256
    base1 = base0 + b[1] * 256
    s2 = base1 + (1 - b[2]) * 128
    base2 = base1 + b[2] * 128
    s3 = base2 + (1 - b[3]) * 64
    base3 = base2 + b[3] * 64

    xv = x_ref[:, :]

    def rope(t, c, s):
        parts = []
        for h in range(HQ):
            lo = t[:, h * DH: h * DH + DH // 2]
            hi = t[:, h * DH + DH // 2: (h + 1) * DH]
            parts.append(-hi)
            parts.append(lo)
        return t * c + jnp.concatenate(parts, axis=1) * s

    kr = rope(mm(xv, wk_ref[:, :]).astype(bf16), cos_ref[:, :], sin_ref[:, :])
    v = mm(xv, wv_ref[:, :]).astype(bf16)

    def partial_rows(row_start, n):
        xr = x_ref[pl.ds(row_start, n), :]
        qb = mm(xr, wq_ref[:, :]).astype(bf16)
        qrb = rope(qb, cos_ref[pl.ds(row_start, n), :],
                   sin_ref[pl.ds(row_start, n), :])
        acch = jnp.zeros((n, D), f32)
        for h in range(HQ):
            hs = slice(h * DH, (h + 1) * DH)
            s = lax.dot_general(
                qrb[:, hs], kr[:, hs], (((1,), (1,)), ((), ())),
                preferred_element_type=f32)
            w = jnp.exp(s * SCALE)
            w = w / jnp.sum(w, axis=-1, keepdims=True)
            ctx = mm(w.astype(bf16), v[:, hs]).astype(bf16)
            acch = acch + mm(ctx, wo_ref[hs, :])
        return acch

    def rs_rdma(r, half, dst):
        return pltpu.make_async_remote_copy(
            src_ref=send_buf.at[pl.ds(0, half), :],
            dst_ref=dst.at[:, :],
            send_sem=rs_send.at[r],
            recv_sem=rs_recv.at[r],
            device_id=rs_partners[r],
            device_id_type=pl.DeviceIdType.MESH,
        )

    blk = partial_rows(send0, 512)
    send_buf[pl.ds(0, 512), :] = blk.astype(bf16)
    r0 = rs_rdma(0, 512, rs_b0)
    r0.start()

    blk1 = partial_rows(s1, 256)
    r0.wait()
    stage = blk1 + rs_b0[pl.ds(s1 - base0, 256), :].astype(f32)
    send_buf[pl.ds(0, 256), :] = stage.astype(bf16)
    r1 = rs_rdma(1, 256, rs_b1)
    r1.start()

    blk2 = partial_rows(s2, 128)
    r1.wait()
    stage = (blk2
             + rs_b0[pl.ds(s2 - base0, 128), :].astype(f32)
             + rs_b1[pl.ds(s2 - base1, 128), :].astype(f32))
    send_buf[pl.ds(0, 128), :] = stage.astype(bf16)
    r2 = rs_rdma(2, 128, rs_b2)
    r2.start()

    dm_buf[:, :] = partial_rows(base2, 128)
    r2.wait()
    stage = (dm_buf[pl.ds(s3 - base2, 64), :]
             + rs_b0[pl.ds(s3 - base0, 64), :].astype(f32)
             + rs_b1[pl.ds(s3 - base1, 64), :].astype(f32)
             + rs_b2[pl.ds(s3 - base2, 64), :].astype(f32))
    send_buf[pl.ds(0, 64), :] = stage.astype(bf16)
    r3 = rs_rdma(3, 64, rs_b3)
    r3.start()
    r3.wait()

    own = (dm_buf[pl.ds(base3 - base2, 64), :]
           + rs_b0[pl.ds(base3 - base0, 64), :].astype(f32)
           + rs_b1[pl.ds(base3 - base1, 64), :].astype(f32)
           + rs_b2[pl.ds(base3 - base2, 64), :].astype(f32)
           + rs_b3[:, :].astype(f32))
    out_ref[pl.ds(base3, 64), :] = own.astype(bf16)

    ag_bit = [_bit(my, bi) for bi in AG_BITS]
    ag_partners = [(lax.bitwise_xor(my, 1 << bi),) for bi in AG_BITS]
    vbase = base3
    rv_start = []
    for j in range(4):
        sz = 64 << j
        rv_start.append(vbase + sz * (1 - 2 * ag_bit[j]))
        vbase = vbase - ag_bit[j] * sz

    def piece_range(p):
        if p == 0:
            return base3, 64
        return rv_start[p - 1], 64 << (p - 1)

    descs = {}
    for k in range(4):
        for p in range(k + 1):
            idx = k * (k + 1) // 2 + p
            start, size = piece_range(p)
            descs[(k, p)] = pltpu.make_async_remote_copy(
                src_ref=out_ref.at[pl.ds(start, size), :],
                dst_ref=out_ref.at[pl.ds(start, size), :],
                send_sem=ag_send.at[idx],
                recv_sem=ag_recv.at[idx],
                device_id=ag_partners[k],
                device_id_type=pl.DeviceIdType.MESH,
            )

    for k in range(4):
        descs[(k, 0)].start()
    for j in range(3):
        for p in range(j + 1):
            descs[(j, p)].wait_recv()
        for k in range(j + 1, 4):
            descs[(k, j + 1)].start()
    for p in range(4):
        descs[(3, p)].wait_recv()
    for k in range(4):
        for p in range(k + 1):
            descs[(k, p)].wait_send()


def kernel(x, Wq, Wk, Wv, Wo):
    bf16 = jnp.bfloat16
    x2 = x.reshape(SQ, D).astype(bf16)

    def perm(w):
        return w.reshape(D, HQ, DH // 2, 2).transpose(0, 1, 3, 2).reshape(D, HQ * DH)

    wq = perm(Wq).astype(bf16)
    wk = perm(Wk).astype(bf16)
    wv = Wv.astype(bf16)
    wo = Wo.astype(bf16)

    inv = 1.0 / (10000.0 ** (jnp.arange(0, DH, 2, dtype=jnp.float32) / DH))
    pos = jnp.arange(SQ, dtype=jnp.float32)[:, None] * inv[None, :]
    cos_h = jnp.concatenate([jnp.cos(pos), jnp.cos(pos)], axis=-1)
    sin_h = jnp.concatenate([jnp.sin(pos), jnp.sin(pos)], axis=-1)
    cos_t = jnp.tile(cos_h, (1, HQ)).astype(bf16)
    sin_t = jnp.tile(sin_h, (1, HQ)).astype(bf16)

    out = pl.pallas_call(
        _body,
        out_shape=jax.ShapeDtypeStruct((SQ, D), bf16),
        in_specs=[pl.BlockSpec(memory_space=pltpu.VMEM)] * 7,
        out_specs=pl.BlockSpec(memory_space=pltpu.VMEM),
        scratch_shapes=[
            pltpu.VMEM((512, D), bf16),
            pltpu.VMEM((128, D), jnp.float32),
            pltpu.VMEM((512, D), bf16),
            pltpu.VMEM((256, D), bf16),
            pltpu.VMEM((128, D), bf16),
            pltpu.VMEM((64, D), bf16),
            pltpu.SemaphoreType.DMA((4,)),
            pltpu.SemaphoreType.DMA((4,)),
            pltpu.SemaphoreType.DMA((10,)),
            pltpu.SemaphoreType.DMA((10,)),
        ],
    )(x2, wq, wk, wv, wo, cos_t, sin_t)
    return out.reshape(1, SQ, D).astype(jnp.float32)


# device time: 95987 ns/iter; 1.1891x vs baseline; 1.0815x over previous
import jax
import jax.numpy as jnp
from jax import lax
from jax.experimental import pallas as pl
from jax.experimental.pallas import tpu as pltpu

N_DEV = 16
SQ = 1024
D = 1024
HQ = 8
DH = 128
SCALE = 0.08838834764831843

RS_BITS = (0, 2, 1, 3)
AG_BITS = (3, 1, 2, 0)


def _bit(my, bi):
    return lax.bitwise_and(lax.shift_right_logical(my, bi), 1)


def _body(x_ref, wq_ref, wk_ref, wv_ref, wo_ref, cos_ref, sin_ref,
          out_ref, acc_ref, send_buf,
          rs_b0, rs_b1, rs_b2, rs_b3,
          rs_send, rs_recv, ag_send, ag_recv):
    my = lax.axis_index("i")

    f32 = jnp.float32
    bf16 = jnp.bfloat16
    mm = lambda a, b: lax.dot_general(
        a, b, (((1,), (0,)), ((), ())), preferred_element_type=f32)

    b = [_bit(my, bi) for bi in RS_BITS]
    rs_partners = [(lax.bitwise_xor(my, 1 << bi),) for bi in RS_BITS]

    barrier = pltpu.get_barrier_semaphore()
    for p in rs_partners:
        pl.semaphore_signal(barrier, inc=1, device_id=p,
                            device_id_type=pl.DeviceIdType.MESH)
    pl.semaphore_wait(barrier, 4)

    send0 = (1 - b[0]) * 512
    base0 = b[0] * 512
    s1 = base0 + (1 - b[1]) * 256
    base1 = base0 + b[1] * 256
    s2 = base1 + (1 - b[2]) * 128
    base2 = base1 + b[2] * 128
    s3 = base2 + (1 - b[3]) * 64
    base3 = base2 + b[3] * 64

    xv = x_ref[:, :]

    def rope(t, c, s):
        parts = []
        for h in range(HQ):
            lo = t[:, h * DH: h * DH + DH // 2]
            hi = t[:, h * DH + DH // 2: (h + 1) * DH]
            parts.append(-hi)
            parts.append(lo)
        return t * c + jnp.concatenate(parts, axis=1) * s

    kr = rope(mm(xv, wk_ref[:, :]).astype(bf16), cos_ref[:, :], sin_ref[:, :])
    v = mm(xv, wv_ref[:, :]).astype(bf16)

    def partial_rows(row_start, n):
        xr = x_ref[pl.ds(row_start, n), :]
        qb = mm(xr, wq_ref[:, :]).astype(bf16)
        qrb = rope(qb, cos_ref[pl.ds(row_start, n), :],
                   sin_ref[pl.ds(row_start, n), :])
        acch = jnp.zeros((n, D), f32)
        for h in range(HQ):
            hs = slice(h * DH, (h + 1) * DH)
            s = lax.dot_general(
                qrb[:, hs], kr[:, hs], (((1,), (1,)), ((), ())),
                preferred_element_type=f32)
            w = jnp.exp(s * SCALE)
            recip = 1.0 / jnp.sum(w, axis=-1, keepdims=True)
            ctx = (mm(w.astype(bf16), v[:, hs]) * recip).astype(bf16)
            acch = acch + mm(ctx, wo_ref[hs, :])
        return acch

    def rs_rdma(r, half, dst):
        return pltpu.make_async_remote_copy(
            src_ref=send_buf.at[pl.ds(0, half), :],
            dst_ref=dst.at[:, :],
            send_sem=rs_send.at[r],
            recv_sem=rs_recv.at[r],
            device_id=rs_partners[r],
            device_id_type=pl.DeviceIdType.MESH,
        )

    blk = partial_rows(send0, 512)
    send_buf[pl.ds(0, 512), :] = blk.astype(bf16)
    r0 = rs_rdma(0, 512, rs_b0)
    r0.start()
    acc_ref[pl.ds(base0, 512), :] = partial_rows(base0, 512)
    r0.wait()

    rs_bufs = [rs_b0, rs_b1, rs_b2, rs_b3]
    base = base0
    prev = rs_b0
    prev_base = base0
    rdmas = []
    for r in range(1, 4):
        half = 512 >> r
        bit = b[r]
        send_start = base + (1 - bit) * half
        keep_start = base + bit * half
        stage = (acc_ref[pl.ds(send_start, half), :]
                 + prev[pl.ds(send_start - prev_base, half), :].astype(f32))
        send_buf[pl.ds(0, half), :] = stage.astype(bf16)
        rd = rs_rdma(r, half, rs_bufs[r])
        rd.start()
        acc_ref[pl.ds(keep_start, half), :] = (
            acc_ref[pl.ds(keep_start, half), :]
            + prev[pl.ds(keep_start - prev_base, half), :].astype(f32))
        rd.wait()
        base = keep_start
        prev = rs_bufs[r]
        prev_base = keep_start
    own = acc_ref[pl.ds(base3, 64), :] + rs_b3[:, :].astype(f32)
    out_ref[pl.ds(base3, 64), :] = own.astype(bf16)

    ag_bit = [_bit(my, bi) for bi in AG_BITS]
    ag_partners = [(lax.bitwise_xor(my, 1 << bi),) for bi in AG_BITS]
    vbase = base3
    rv_start = []
    for j in range(4):
        sz = 64 << j
        rv_start.append(vbase + sz * (1 - 2 * ag_bit[j]))
        vbase = vbase - ag_bit[j] * sz

    def piece_range(p):
        if p == 0:
            return base3, 64
        return rv_start[p - 1], 64 << (p - 1)

    descs = {}
    for k in range(4):
        for p in range(k + 1):
            idx = k * (k + 1) // 2 + p
            start, size = piece_range(p)
            descs[(k, p)] = pltpu.make_async_remote_copy(
                src_ref=out_ref.at[pl.ds(start, size), :],
                dst_ref=out_ref.at[pl.ds(start, size), :],
                send_sem=ag_send.at[idx],
                recv_sem=ag_recv.at[idx],
                device_id=ag_partners[k],
                device_id_type=pl.DeviceIdType.MESH,
            )

    for k in range(4):
        descs[(k, 0)].start()
    for j in range(3):
        for p in range(j + 1):
            descs[(j, p)].wait_recv()
        for k in range(j + 1, 4):
            descs[(k, j + 1)].start()
    for p in range(4):
        descs[(3, p)].wait_recv()
    for k in range(4):
        for p in range(k + 1):
            descs[(k, p)].wait_send()


def kernel(x, Wq, Wk, Wv, Wo):
    bf16 = jnp.bfloat16
    x2 = x.reshape(SQ, D).astype(bf16)

    def perm(w):
        return w.reshape(D, HQ, DH // 2, 2).transpose(0, 1, 3, 2).reshape(D, HQ * DH)

    wq = perm(Wq).astype(bf16)
    wk = perm(Wk).astype(bf16)
    wv = Wv.astype(bf16)
    wo = Wo.astype(bf16)

    inv = 1.0 / (10000.0 ** (jnp.arange(0, DH, 2, dtype=jnp.float32) / DH))
    pos = jnp.arange(SQ, dtype=jnp.float32)[:, None] * inv[None, :]
    cos_h = jnp.concatenate([jnp.cos(pos), jnp.cos(pos)], axis=-1)
    sin_h = jnp.concatenate([jnp.sin(pos), jnp.sin(pos)], axis=-1)
    cos_t = jnp.tile(cos_h, (1, HQ)).astype(bf16)
    sin_t = jnp.tile(sin_h, (1, HQ)).astype(bf16)

    out = pl.pallas_call(
        _body,
        out_shape=jax.ShapeDtypeStruct((SQ, D), bf16),
        in_specs=[pl.BlockSpec(memory_space=pltpu.VMEM)] * 7,
        out_specs=pl.BlockSpec(memory_space=pltpu.VMEM),
        scratch_shapes=[
            pltpu.VMEM((SQ, D), jnp.float32),
            pltpu.VMEM((512, D), bf16),
            pltpu.VMEM((512, D), bf16),
            pltpu.VMEM((256, D), bf16),
            pltpu.VMEM((128, D), bf16),
            pltpu.VMEM((64, D), bf16),
            pltpu.SemaphoreType.DMA((4,)),
            pltpu.SemaphoreType.DMA((4,)),
            pltpu.SemaphoreType.DMA((10,)),
            pltpu.SemaphoreType.DMA((10,)),
        ],
        compiler_params=pltpu.CompilerParams(collective_id=0),
    )(x2, wq, wk, wv, wo, cos_t, sin_t)
    return out.reshape(1, SQ, D).astype(jnp.float32)


# device time: 95986 ns/iter; 1.1892x vs baseline; 1.0000x over previous
import jax
import jax.numpy as jnp
from jax import lax
from jax.experimental import pallas as pl
from jax.experimental.pallas import tpu as pltpu

N_DEV = 16
SQ = 1024
D = 1024
HQ = 8
DH = 128
SCALE = 0.08838834764831843

RS_BITS = (0, 2, 1, 3)
AG_BITS = (3, 1, 2, 0)


def _bit(my, bi):
    return lax.bitwise_and(lax.shift_right_logical(my, bi), 1)


def _body(x_ref, wq_ref, wk_ref, wv_ref, wo_ref, cos_ref, sin_ref,
          out_ref, acc_ref, send_buf,
          rs_b0, rs_b1, rs_b2, rs_b3,
          rs_send, rs_recv, ag_send, ag_recv):
    my = lax.axis_index("i")

    f32 = jnp.float32
    bf16 = jnp.bfloat16
    mm = lambda a, b: lax.dot_general(
        a, b, (((1,), (0,)), ((), ())), preferred_element_type=f32)

    b = [_bit(my, bi) for bi in RS_BITS]
    rs_partners = [(lax.bitwise_xor(my, 1 << bi),) for bi in RS_BITS]

    barrier = pltpu.get_barrier_semaphore()
    for p in rs_partners:
        pl.semaphore_signal(barrier, inc=1, device_id=p,
                            device_id_type=pl.DeviceIdType.MESH)
    pl.semaphore_wait(barrier, 4)

    send0 = (1 - b[0]) * 512
    base0 = b[0] * 512
    s1 = base0 + (1 - b[1]) * 256
    base1 = base0 + b[1] * 256
    s2 = base1 + (1 - b[2]) * 128
    base2 = base1 + b[2] * 128
    s3 = base2 + (1 - b[3]) * 64
    base3 = base2 + b[3] * 64

    xv = x_ref[:, :]

    def rope(t, c, s):
        parts = []
        for h in range(HQ):
            lo = t[:, h * DH: h * DH + DH // 2]
            hi = t[:, h * DH + DH // 2: (h + 1) * DH]
            parts.append(-hi)
            parts.append(lo)
        return t * c + jnp.concatenate(parts, axis=1) * s

    kr = rope(mm(xv, wk_ref[:, :]).astype(bf16), cos_ref[:, :], sin_ref[:, :])
    v = mm(xv, wv_ref[:, :]).astype(bf16)

    def partial_rows(row_start, n):
        xr = x_ref[pl.ds(row_start, n), :]
        qb = mm(xr, wq_ref[:, :]).astype(bf16)
        qrb = rope(qb, cos_ref[pl.ds(row_start, n), :],
                   sin_ref[pl.ds(row_start, n), :])
        acch = jnp.zeros((n, D), f32)
        for h in range(HQ):
            hs = slice(h * DH, (h + 1) * DH)
            s = lax.dot_general(
                qrb[:, hs], kr[:, hs], (((1,), (1,)), ((), ())),
                preferred_element_type=f32)
            w = jnp.exp(s * SCALE)
            recip = 1.0 / jnp.sum(w, axis=-1, keepdims=True)
            ctx = (mm(w.astype(bf16), v[:, hs]) * recip).astype(bf16)
            acch = acch + mm(ctx, wo_ref[hs, :])
        return acch

    def rs_rdma(r, half, dst):
        return pltpu.make_async_remote_copy(
            src_ref=send_buf.at[pl.ds(0, half), :],
            dst_ref=dst.at[:, :],
            send_sem=rs_send.at[r],
            recv_sem=rs_recv.at[r],
            device_id=rs_partners[r],
            device_id_type=pl.DeviceIdType.MESH,
        )

    blk = partial_rows(send0, 512)
    send_buf[pl.ds(0, 512), :] = blk.astype(bf16)
    r0 = rs_rdma(0, 512, rs_b0)
    r0.start()
    blk1 = partial_rows(s1, 256)
    r0.wait()

    stage = blk1 + rs_b0[pl.ds(s1 - base0, 256), :].astype(f32)
    send_buf[pl.ds(0, 256), :] = stage.astype(bf16)
    r1 = rs_rdma(1, 256, rs_b1)
    r1.start()
    acc_ref[:, :] = partial_rows(base1, 256)
    r1.wait()

    stage = (acc_ref[pl.ds(s2 - base1, 128), :]
             + rs_b0[pl.ds(s2 - base0, 128), :].astype(f32)
             + rs_b1[pl.ds(s2 - base1, 128), :].astype(f32))
    send_buf[pl.ds(0, 128), :] = stage.astype(bf16)
    r2 = rs_rdma(2, 128, rs_b2)
    r2.start()
    acc_ref[pl.ds(base2 - base1, 128), :] = (
        acc_ref[pl.ds(base2 - base1, 128), :]
        + rs_b0[pl.ds(base2 - base0, 128), :].astype(f32)
        + rs_b1[pl.ds(base2 - base1, 128), :].astype(f32))
    r2.wait()

    stage = (acc_ref[pl.ds(s3 - base1, 64), :]
             + rs_b2[pl.ds(s3 - base2, 64), :].astype(f32))
    send_buf[pl.ds(0, 64), :] = stage.astype(bf16)
    r3 = rs_rdma(3, 64, rs_b3)
    r3.start()
    own_pre = (acc_ref[pl.ds(base3 - base1, 64), :]
               + rs_b2[pl.ds(base3 - base2, 64), :].astype(f32))
    r3.wait()
    out_ref[pl.ds(base3, 64), :] = (
        own_pre + rs_b3[:, :].astype(f32)).astype(bf16)

    ag_bit = [_bit(my, bi) for bi in AG_BITS]
    ag_partners = [(lax.bitwise_xor(my, 1 << bi),) for bi in AG_BITS]
    vbase = base3
    rv_start = []
    for j in range(4):
        sz = 64 << j
        rv_start.append(vbase + sz * (1 - 2 * ag_bit[j]))
        vbase = vbase - ag_bit[j] * sz

    def piece_range(p):
        if p == 0:
            return base3, 64
        return rv_start[p - 1], 64 << (p - 1)

    descs = {}
    for k in range(4):
        for p in range(k + 1):
            idx = k * (k + 1) // 2 + p
            start, size = piece_range(p)
            descs[(k, p)] = pltpu.make_async_remote_copy(
                src_ref=out_ref.at[pl.ds(start, size), :],
                dst_ref=out_ref.at[pl.ds(start, size), :],
                send_sem=ag_send.at[idx],
                recv_sem=ag_recv.at[idx],
                device_id=ag_partners[k],
                device_id_type=pl.DeviceIdType.MESH,
            )

    for k in range(4):
        descs[(k, 0)].start()
    for j in range(3):
        for p in range(j + 1):
            descs[(j, p)].wait_recv()
        for k in range(j + 1, 4):
            descs[(k, j + 1)].start()
    for p in range(4):
        descs[(3, p)].wait_recv()
    for k in range(4):
        for p in range(k + 1):
            descs[(k, p)].wait_send()


def kernel(x, Wq, Wk, Wv, Wo):
    bf16 = jnp.bfloat16
    x2 = x.reshape(SQ, D).astype(bf16)

    def perm(w):
        return w.reshape(D, HQ, DH // 2, 2).transpose(0, 1, 3, 2).reshape(D, HQ * DH)

    wq = perm(Wq).astype(bf16)
    wk = perm(Wk).astype(bf16)
    wv = Wv.astype(bf16)
    wo = Wo.astype(bf16)

    inv = 1.0 / (10000.0 ** (jnp.arange(0, DH, 2, dtype=jnp.float32) / DH))
    pos = jnp.arange(SQ, dtype=jnp.float32)[:, None] * inv[None, :]
    cos_h = jnp.concatenate([jnp.cos(pos), jnp.cos(pos)], axis=-1)
    sin_h = jnp.concatenate([jnp.sin(pos), jnp.sin(pos)], axis=-1)
    cos_t = jnp.tile(cos_h, (1, HQ)).astype(bf16)
    sin_t = jnp.tile(sin_h, (1, HQ)).astype(bf16)

    out = pl.pallas_call(
        _body,
        out_shape=jax.ShapeDtypeStruct((SQ, D), bf16),
        in_specs=[pl.BlockSpec(memory_space=pltpu.VMEM)] * 7,
        out_specs=pl.BlockSpec(memory_space=pltpu.VMEM),
        scratch_shapes=[
            pltpu.VMEM((256, D), jnp.float32),
            pltpu.VMEM((512, D), bf16),
            pltpu.VMEM((512, D), bf16),
            pltpu.VMEM((256, D), bf16),
            pltpu.VMEM((128, D), bf16),
            pltpu.VMEM((64, D), bf16),
            pltpu.SemaphoreType.DMA((4,)),
            pltpu.SemaphoreType.DMA((4,)),
            pltpu.SemaphoreType.DMA((10,)),
            pltpu.SemaphoreType.DMA((10,)),
        ],
        compiler_params=pltpu.CompilerParams(collective_id=0),
    )(x2, wq, wk, wv, wo, cos_t, sin_t)
    return out.reshape(1, SQ, D).astype(jnp.float32)


# device time: 92724 ns/iter; 1.2310x vs baseline; 1.0352x over previous
import jax
import jax.numpy as jnp
from jax import lax
from jax.experimental import pallas as pl
from jax.experimental.pallas import tpu as pltpu

N_DEV = 16
SQ = 1024
D = 1024
HQ = 8
DH = 128
SCALE = 0.08838834764831843

RS_BITS = (0, 2, 1, 3)
AG_BITS = (3, 1, 2, 0)


def _bit(my, bi):
    return lax.bitwise_and(lax.shift_right_logical(my, bi), 1)


def _body(x_ref, wq_ref, wk_ref, wv_ref, wo_ref, cos_ref, sin_ref,
          out_ref, acc_ref, send_buf,
          rs_b0, rs_b1, rs_b2, rs_b3,
          rs_send, rs_recv, ag_send, ag_recv):
    my = lax.axis_index("i")

    f32 = jnp.float32
    bf16 = jnp.bfloat16
    mm = lambda a, b: lax.dot_general(
        a, b, (((1,), (0,)), ((), ())), preferred_element_type=f32)

    b = [_bit(my, bi) for bi in RS_BITS]
    rs_partners = [(lax.bitwise_xor(my, 1 << bi),) for bi in RS_BITS]

    barrier = pltpu.get_barrier_semaphore()
    for p in rs_partners:
        pl.semaphore_signal(barrier, inc=1, device_id=p,
                            device_id_type=pl.DeviceIdType.MESH)
    pl.semaphore_wait(barrier, 4)

    send0 = (1 - b[0]) * 512
    base0 = b[0] * 512
    s1 = base0 + (1 - b[1]) * 256
    base1 = base0 + b[1] * 256
    s2 = base1 + (1 - b[2]) * 128
    base2 = base1 + b[2] * 128
    s3 = base2 + (1 - b[3]) * 64
    base3 = base2 + b[3] * 64

    xv = x_ref[:, :]

    def rope(t, c, s):
        parts = []
        for h in range(HQ):
            th = t[:, h * DH: (h + 1) * DH]
            lo = th[:, : DH // 2]
            hi = th[:, DH // 2:]
            rot = jnp.concatenate([-hi, lo], axis=1)
            parts.append(th * c + rot * s)
        return jnp.concatenate(parts, axis=1)

    kr = rope(mm(xv, wk_ref[:, :]).astype(bf16), cos_ref[:, :], sin_ref[:, :])
    v = mm(xv, wv_ref[:, :]).astype(bf16)

    def partial_rows(row_start, n):
        xr = x_ref[pl.ds(row_start, n), :]
        qb = mm(xr, wq_ref[:, :]).astype(bf16)
        qrb = rope(qb, cos_ref[pl.ds(row_start, n), :],
                   sin_ref[pl.ds(row_start, n), :])
        acch = jnp.zeros((n, D), f32)
        for h in range(HQ):
            hs = slice(h * DH, (h + 1) * DH)
            s = lax.dot_general(
                qrb[:, hs], kr[:, hs], (((1,), (1,)), ((), ())),
                preferred_element_type=f32)
            w = jnp.exp(s * SCALE)
            recip = 1.0 / jnp.sum(w, axis=-1, keepdims=True)
            ctx = (mm(w.astype(bf16), v[:, hs]) * recip).astype(bf16)
            acch = acch + mm(ctx, wo_ref[hs, :])
        return acch

    def rs_rdma(r, half, dst):
        return pltpu.make_async_remote_copy(
            src_ref=send_buf.at[pl.ds(0, half), :],
            dst_ref=dst.at[:, :],
            send_sem=rs_send.at[r],
            recv_sem=rs_recv.at[r],
            device_id=rs_partners[r],
            device_id_type=pl.DeviceIdType.MESH,
        )

    blk = partial_rows(send0, 512)
    send_buf[pl.ds(0, 512), :] = blk.astype(bf16)
    r0 = rs_rdma(0, 512, rs_b0)
    r0.start()
    blk1 = partial_rows(s1, 256)
    r0.wait()

    stage = blk1 + rs_b0[pl.ds(s1 - base0, 256), :].astype(f32)
    send_buf[pl.ds(0, 256), :] = stage.astype(bf16)
    r1 = rs_rdma(1, 256, rs_b1)
    r1.start()
    acc_ref[:, :] = partial_rows(base1, 256)
    r1.wait()

    stage = (acc_ref[pl.ds(s2 - base1, 128), :]
             + rs_b0[pl.ds(s2 - base0, 128), :].astype(f32)
             + rs_b1[pl.ds(s2 - base1, 128), :].astype(f32))
    send_buf[pl.ds(0, 128), :] = stage.astype(bf16)
    r2 = rs_rdma(2, 128, rs_b2)
    r2.start()
    acc_ref[pl.ds(base2 - base1, 128), :] = (
        acc_ref[pl.ds(base2 - base1, 128), :]
        + rs_b0[pl.ds(base2 - base0, 128), :].astype(f32)
        + rs_b1[pl.ds(base2 - base1, 128), :].astype(f32))
    r2.wait()

    stage = (acc_ref[pl.ds(s3 - base1, 64), :]
             + rs_b2[pl.ds(s3 - base2, 64), :].astype(f32))
    send_buf[pl.ds(0, 64), :] = stage.astype(bf16)
    r3 = rs_rdma(3, 64, rs_b3)
    r3.start()
    own_pre = (acc_ref[pl.ds(base3 - base1, 64), :]
               + rs_b2[pl.ds(base3 - base2, 64), :].astype(f32))
    r3.wait()
    out_ref[pl.ds(base3, 64), :] = (
        own_pre + rs_b3[:, :].astype(f32)).astype(bf16)

    ag_bit = [_bit(my, bi) for bi in AG_BITS]
    ag_partners = [(lax.bitwise_xor(my, 1 << bi),) for bi in AG_BITS]
    vbase = base3
    rv_start = []
    for j in range(4):
        sz = 64 << j
        rv_start.append(vbase + sz * (1 - 2 * ag_bit[j]))
        vbase = vbase - ag_bit[j] * sz

    def piece_range(p):
        if p == 0:
            return base3, 64
        return rv_start[p - 1], 64 << (p - 1)

    descs = {}
    for k in range(4):
        for p in range(k + 1):
            idx = k * (k + 1) // 2 + p
            start, size = piece_range(p)
            descs[(k, p)] = pltpu.make_async_remote_copy(
                src_ref=out_ref.at[pl.ds(start, size), :],
                dst_ref=out_ref.at[pl.ds(start, size), :],
                send_sem=ag_send.at[idx],
                recv_sem=ag_recv.at[idx],
                device_id=ag_partners[k],
                device_id_type=pl.DeviceIdType.MESH,
            )

    for k in range(4):
        descs[(k, 0)].start()
    for j in range(3):
        for p in range(j + 1):
            descs[(j, p)].wait_recv()
        for k in range(j + 1, 4):
            descs[(k, j + 1)].start()
    for p in range(4):
        descs[(3, p)].wait_recv()
    for k in range(4):
        for p in range(k + 1):
            descs[(k, p)].wait_send()


def kernel(x, Wq, Wk, Wv, Wo):
    bf16 = jnp.bfloat16
    x2 = x.reshape(SQ, D).astype(bf16)

    def perm(w):
        return (w.astype(bf16).reshape(D, HQ, DH // 2, 2)
                .transpose(0, 1, 3, 2).reshape(D, HQ * DH))

    wq = perm(Wq)
    wk = perm(Wk)
    wv = Wv.astype(bf16)
    wo = Wo.astype(bf16)

    inv = 1.0 / (10000.0 ** (jnp.arange(0, DH, 2, dtype=jnp.float32) / DH))
    pos = jnp.arange(SQ, dtype=jnp.float32)[:, None] * inv[None, :]
    cos_t = jnp.concatenate([jnp.cos(pos)] * 2, axis=-1).astype(bf16)
    sin_t = jnp.concatenate([jnp.sin(pos)] * 2, axis=-1).astype(bf16)

    out = pl.pallas_call(
        _body,
        out_shape=jax.ShapeDtypeStruct((SQ, D), bf16),
        in_specs=[pl.BlockSpec(memory_space=pltpu.VMEM)] * 7,
        out_specs=pl.BlockSpec(memory_space=pltpu.VMEM),
        scratch_shapes=[
            pltpu.VMEM((256, D), jnp.float32),
            pltpu.VMEM((512, D), bf16),
            pltpu.VMEM((512, D), bf16),
            pltpu.VMEM((256, D), bf16),
            pltpu.VMEM((128, D), bf16),
            pltpu.VMEM((64, D), bf16),
            pltpu.SemaphoreType.DMA((4,)),
            pltpu.SemaphoreType.DMA((4,)),
            pltpu.SemaphoreType.DMA((10,)),
            pltpu.SemaphoreType.DMA((10,)),
        ],
        compiler_params=pltpu.CompilerParams(collective_id=0),
    )(x2, wq, wk, wv, wo, cos_t, sin_t)
    return out.reshape(1, SQ, D)


# device time: 92380 ns/iter; 1.2356x vs baseline; 1.0037x over previous
import jax
import jax.numpy as jnp
from jax import lax
from jax.experimental import pallas as pl
from jax.experimental.pallas import tpu as pltpu

N_DEV = 16
SQ = 1024
D = 1024
HQ = 8
DH = 128
SCALE = 0.08838834764831843

RS_BITS = (0, 2, 1, 3)
AG_BITS = (3, 1, 2, 0)


def _bit(my, bi):
    return lax.bitwise_and(lax.shift_right_logical(my, bi), 1)


def _body(x_ref, wq_ref, wk_ref, wv_ref, wo_ref, cos_ref, sin_ref,
          out_ref, acc_ref, send_buf,
          rs_b0, rs_b1, rs_b2, rs_b3,
          rs_send, rs_recv, ag_send, ag_recv):
    my = lax.axis_index("i")

    f32 = jnp.float32
    bf16 = jnp.bfloat16
    mm = lambda a, b: lax.dot_general(
        a, b, (((1,), (0,)), ((), ())), preferred_element_type=f32)

    b = [_bit(my, bi) for bi in RS_BITS]
    rs_partners = [(lax.bitwise_xor(my, 1 << bi),) for bi in RS_BITS]

    barrier = pltpu.get_barrier_semaphore()
    for p in rs_partners:
        pl.semaphore_signal(barrier, inc=1, device_id=p,
                            device_id_type=pl.DeviceIdType.MESH)
    pl.semaphore_wait(barrier, 4)

    send0 = (1 - b[0]) * 512
    base0 = b[0] * 512
    s1 = base0 + (1 - b[1]) * 256
    base1 = base0 + b[1] * 256
    s2 = base1 + (1 - b[2]) * 128
    base2 = base1 + b[2] * 128
    s3 = base2 + (1 - b[3]) * 64
    base3 = base2 + b[3] * 64

    xv = x_ref[:, :]

    def rope(t, c, s):
        parts = []
        for h in range(HQ):
            th = t[:, h * DH: (h + 1) * DH]
            lo = th[:, : DH // 2]
            hi = th[:, DH // 2:]
            rot = jnp.concatenate([-hi, lo], axis=1)
            parts.append(th * c + rot * s)
        return jnp.concatenate(parts, axis=1)

    kr = rope(mm(xv, wk_ref[:, :]).astype(bf16), cos_ref[:, :], sin_ref[:, :])
    v = mm(xv, wv_ref[:, :]).astype(bf16)

    def partial_rows(row_start, n):
        xr = x_ref[pl.ds(row_start, n), :]
        qb = (mm(xr, wq_ref[:, :]) * SCALE).astype(bf16)
        qrb = rope(qb, cos_ref[pl.ds(row_start, n), :],
                   sin_ref[pl.ds(row_start, n), :])
        acch = jnp.zeros((n, D), f32)
        for h in range(HQ):
            hs = slice(h * DH, (h + 1) * DH)
            s = lax.dot_general(
                qrb[:, hs], kr[:, hs], (((1,), (1,)), ((), ())),
                preferred_element_type=f32)
            w = jnp.exp(s)
            recip = 1.0 / jnp.sum(w, axis=-1, keepdims=True)
            ctx = (mm(w.astype(bf16), v[:, hs]) * recip).astype(bf16)
            acch = acch + mm(ctx, wo_ref[hs, :])
        return acch

    def rs_rdma(r, half, dst):
        return pltpu.make_async_remote_copy(
            src_ref=send_buf.at[pl.ds(0, half), :],
            dst_ref=dst.at[:, :],
            send_sem=rs_send.at[r],
            recv_sem=rs_recv.at[r],
            device_id=rs_partners[r],
            device_id_type=pl.DeviceIdType.MESH,
        )

    blk = partial_rows(send0, 512)
    send_buf[pl.ds(0, 512), :] = blk.astype(bf16)
    r0 = rs_rdma(0, 512, rs_b0)
    r0.start()
    blk1 = partial_rows(s1, 256)
    r0.wait()

    stage = blk1 + rs_b0[pl.ds(s1 - base0, 256), :].astype(f32)
    send_buf[pl.ds(0, 256), :] = stage.astype(bf16)
    r1 = rs_rdma(1, 256, rs_b1)
    r1.start()
    acc_ref[:, :] = partial_rows(base1, 256)
    r1.wait()

    stage = (acc_ref[pl.ds(s2 - base1, 128), :]
             + rs_b0[pl.ds(s2 - base0, 128), :].astype(f32)
             + rs_b1[pl.ds(s2 - base1, 128), :].astype(f32))
    send_buf[pl.ds(0, 128), :] = stage.astype(bf16)
    r2 = rs_rdma(2, 128, rs_b2)
    r2.start()
    acc_ref[pl.ds(base2 - base1, 128), :] = (
        acc_ref[pl.ds(base2 - base1, 128), :]
        + rs_b0[pl.ds(base2 - base0, 128), :].astype(f32)
        + rs_b1[pl.ds(base2 - base1, 128), :].astype(f32))
    r2.wait()

    stage = (acc_ref[pl.ds(s3 - base1, 64), :]
             + rs_b2[pl.ds(s3 - base2, 64), :].astype(f32))
    send_buf[pl.ds(0, 64), :] = stage.astype(bf16)
    r3 = rs_rdma(3, 64, rs_b3)
    r3.start()
    own_pre = (acc_ref[pl.ds(base3 - base1, 64), :]
               + rs_b2[pl.ds(base3 - base2, 64), :].astype(f32))
    r3.wait()
    out_ref[pl.ds(base3, 64), :] = (
        own_pre + rs_b3[:, :].astype(f32)).astype(bf16)

    ag_bit = [_bit(my, bi) for bi in AG_BITS]
    ag_partners = [(lax.bitwise_xor(my, 1 << bi),) for bi in AG_BITS]
    vbase = base3
    rv_start = []
    for j in range(4):
        sz = 64 << j
        rv_start.append(vbase + sz * (1 - 2 * ag_bit[j]))
        vbase = vbase - ag_bit[j] * sz

    def piece_range(p):
        if p == 0:
            return base3, 64
        return rv_start[p - 1], 64 << (p - 1)

    descs = {}
    for k in range(4):
        for p in range(k + 1):
            idx = k * (k + 1) // 2 + p
            start, size = piece_range(p)
            descs[(k, p)] = pltpu.make_async_remote_copy(
                src_ref=out_ref.at[pl.ds(start, size), :],
                dst_ref=out_ref.at[pl.ds(start, size), :],
                send_sem=ag_send.at[idx],
                recv_sem=ag_recv.at[idx],
                device_id=ag_partners[k],
                device_id_type=pl.DeviceIdType.MESH,
            )

    for k in range(4):
        descs[(k, 0)].start()
    for j in range(3):
        for p in range(j + 1):
            descs[(j, p)].wait_recv()
        for k in range(j + 1, 4):
            descs[(k, j + 1)].start()
    for p in range(4):
        descs[(3, p)].wait_recv()
    for k in range(4):
        for p in range(k + 1):
            descs[(k, p)].wait_send()


def kernel(x, Wq, Wk, Wv, Wo):
    bf16 = jnp.bfloat16
    x2 = x.reshape(SQ, D).astype(bf16)

    def perm(w):
        return (w.astype(bf16).reshape(D, HQ, DH // 2, 2)
                .transpose(0, 1, 3, 2).reshape(D, HQ * DH))

    wq = perm(Wq)
    wk = perm(Wk)
    wv = Wv.astype(bf16)
    wo = Wo.astype(bf16)

    inv = 1.0 / (10000.0 ** (jnp.arange(0, DH, 2, dtype=jnp.float32) / DH))
    pos = jnp.arange(SQ, dtype=jnp.float32)[:, None] * inv[None, :]
    cos_t = jnp.concatenate([jnp.cos(pos)] * 2, axis=-1).astype(bf16)
    sin_t = jnp.concatenate([jnp.sin(pos)] * 2, axis=-1).astype(bf16)

    out = pl.pallas_call(
        _body,
        out_shape=jax.ShapeDtypeStruct((SQ, D), bf16),
        in_specs=[pl.BlockSpec(memory_space=pltpu.VMEM)] * 7,
        out_specs=pl.BlockSpec(memory_space=pltpu.VMEM),
        scratch_shapes=[
            pltpu.VMEM((256, D), jnp.float32),
            pltpu.VMEM((512, D), bf16),
            pltpu.VMEM((512, D), bf16),
            pltpu.VMEM((256, D), bf16),
            pltpu.VMEM((128, D), bf16),
            pltpu.VMEM((64, D), bf16),
            pltpu.SemaphoreType.DMA((4,)),
            pltpu.SemaphoreType.DMA((4,)),
            pltpu.SemaphoreType.DMA((10,)),
            pltpu.SemaphoreType.DMA((10,)),
        ],
        compiler_params=pltpu.CompilerParams(collective_id=0),
    )(x2, wq, wk, wv, wo, cos_t, sin_t)
    return out.reshape(1, SQ, D)
